# initial kernel scaffold (unmeasured)
import jax
import jax.numpy as jnp
from jax import lax
from jax.experimental import pallas as pl
from jax.experimental.pallas import tpu as pltpu

N_DEV = 4


def kernel(A, B):
    A = A.astype(jnp.bfloat16)
    B = B.astype(jnp.bfloat16)
    m, _ = A.shape
    _, n = B.shape
    m_blk = m // N_DEV

    def body(a_ref, b_ref, out_ref, comm_ref, send_sems, recv_sems):
        my = lax.axis_index("i")
        left = lax.rem(my + N_DEV - 1, N_DEV)
        right = lax.rem(my + 1, N_DEV)

        barrier_sem = pltpu.get_barrier_semaphore()
        for nbr in (left, right):
            pl.semaphore_signal(
                barrier_sem, inc=1,
                device_id=(nbr,), device_id_type=pl.DeviceIdType.MESH,
            )
        pl.semaphore_wait(barrier_sem, 2)

        def partial(c):
            a_blk = a_ref[pl.ds(c * m_blk, m_blk), :]
            return jnp.dot(a_blk, b_ref[:, :], preferred_element_type=jnp.float32)

        comm_ref[2, :, :] = partial(lax.rem(my + 3, N_DEV)).astype(jnp.bfloat16)

        send_slots = (2, 0, 1)
        for s in range(N_DEV - 1):
            rdma = pltpu.make_async_remote_copy(
                src_ref=comm_ref.at[send_slots[s]],
                dst_ref=comm_ref.at[s],
                send_sem=send_sems.at[s],
                recv_sem=recv_sems.at[s],
                device_id=(right,),
                device_id_type=pl.DeviceIdType.MESH,
            )
            rdma.start()
            rdma.wait()

            c = lax.rem(my + (N_DEV - 2 - s), N_DEV)
            acc = partial(c) + comm_ref[s, :, :].astype(jnp.float32)
            if s < N_DEV - 2:
                comm_ref[s, :, :] = acc.astype(jnp.bfloat16)
            else:
                out_ref[:, :] = acc

    return pl.pallas_call(
        body,
        out_shape=jax.ShapeDtypeStruct((m_blk, n), jnp.float32),
        in_specs=[
            pl.BlockSpec(memory_space=pltpu.VMEM),
            pl.BlockSpec(memory_space=pltpu.VMEM),
        ],
        out_specs=pl.BlockSpec(memory_space=pltpu.VMEM),
        scratch_shapes=[
            pltpu.VMEM((3, m_blk, n), jnp.bfloat16),
            pltpu.SemaphoreType.DMA((3,)),
            pltpu.SemaphoreType.DMA((3,)),
        ],
        compiler_params=pltpu.CompilerParams(collective_id=0),
    )(A, B)


# baseline (device time: 407358 ns/iter reference)
import jax
import jax.numpy as jnp
from jax import lax
from jax.experimental import pallas as pl
from jax.experimental.pallas import tpu as pltpu

N_DEV = 4


def kernel(A, B):
    A = A.astype(jnp.bfloat16)
    B = B.astype(jnp.bfloat16)
    m, _ = A.shape
    _, n = B.shape
    m_blk = m // N_DEV

    def body(a_ref, b_ref, out_ref, comm_ref, send_sems, recv_sems):
        my = lax.axis_index("i")
        left = lax.rem(my + N_DEV - 1, N_DEV)
        right = lax.rem(my + 1, N_DEV)

        barrier_sem = pltpu.get_barrier_semaphore()
        for nbr in (left, right):
            pl.semaphore_signal(
                barrier_sem, inc=1,
                device_id=(nbr,), device_id_type=pl.DeviceIdType.MESH,
            )
        pl.semaphore_wait(barrier_sem, 2)

        def partial(c):
            a_blk = a_ref[pl.ds(c * m_blk, m_blk), :]
            return jnp.dot(a_blk, b_ref[:, :], preferred_element_type=jnp.float32)

        comm_ref[0, :, :] = partial(lax.rem(my + 3, N_DEV)).astype(jnp.bfloat16)

        for s in range(N_DEV - 1):
            send_slot = s % 2
            recv_slot = (s + 1) % 2
            rdma = pltpu.make_async_remote_copy(
                src_ref=comm_ref.at[send_slot],
                dst_ref=comm_ref.at[recv_slot],
                send_sem=send_sems.at[s],
                recv_sem=recv_sems.at[s],
                device_id=(right,),
                device_id_type=pl.DeviceIdType.MESH,
            )
            rdma.start()
            rdma.wait()

            c = lax.rem(my + (N_DEV - 2 - s), N_DEV)
            acc = partial(c) + comm_ref[recv_slot, :, :].astype(jnp.float32)
            if s < N_DEV - 2:
                comm_ref[recv_slot, :, :] = acc.astype(jnp.bfloat16)
            else:
                out_ref[:, :] = acc.astype(jnp.bfloat16)

    return pl.pallas_call(
        body,
        out_shape=jax.ShapeDtypeStruct((m_blk, n), jnp.bfloat16),
        in_specs=[
            pl.BlockSpec(memory_space=pltpu.VMEM),
            pl.BlockSpec(memory_space=pltpu.VMEM),
        ],
        out_specs=pl.BlockSpec(memory_space=pltpu.VMEM),
        scratch_shapes=[
            pltpu.VMEM((2, m_blk, n), jnp.bfloat16),
            pltpu.SemaphoreType.DMA((3,)),
            pltpu.SemaphoreType.DMA((3,)),
        ],
        compiler_params=pltpu.CompilerParams(
            collective_id=0,
            vmem_limit_bytes=100 * 1024 * 1024,
        ),
    )(A, B)


# device time: 218407 ns/iter; 1.8651x vs baseline; 1.8651x over previous
import jax
import jax.numpy as jnp
from jax import lax
from jax.experimental import pallas as pl
from jax.experimental.pallas import tpu as pltpu

N_DEV = 4


def kernel(A, B):
    A = A.astype(jnp.bfloat16)
    B = B.astype(jnp.bfloat16)
    m, _ = A.shape
    _, n = B.shape
    m_blk = m // N_DEV
    h = n // 2

    A_SLOT = (0, 2, 1)
    B_SLOT = (1, 0, 2)
    C_SLOT = (2, 1, 0)

    def body(a_ref, b_ref, out_ref,
             comm_r, comm_l,
             send_r, recv_r, send_l, recv_l, out_sems):
        my = lax.axis_index("i")
        left = lax.rem(my + N_DEV - 1, N_DEV)
        right = lax.rem(my + 1, N_DEV)

        barrier_sem = pltpu.get_barrier_semaphore()
        for nbr in (left, right):
            pl.semaphore_signal(
                barrier_sem, inc=1,
                device_id=(nbr,), device_id_type=pl.DeviceIdType.MESH,
            )
        pl.semaphore_wait(barrier_sem, 2)

        def a_blk(c):
            return a_ref[pl.ds(lax.rem(c, N_DEV) * m_blk, m_blk), :]

        def partial_r(c):
            return jnp.dot(a_blk(c), b_ref[:, 0:h],
                           preferred_element_type=jnp.float32)

        def partial_l(c):
            return jnp.dot(a_blk(c), b_ref[:, h:n],
                           preferred_element_type=jnp.float32)

        def hop_rdmas(s):
            r = pltpu.make_async_remote_copy(
                src_ref=comm_r.at[A_SLOT[s]],
                dst_ref=comm_r.at[B_SLOT[s]],
                send_sem=send_r.at[s],
                recv_sem=recv_r.at[s],
                device_id=(right,),
                device_id_type=pl.DeviceIdType.MESH,
            )
            l = pltpu.make_async_remote_copy(
                src_ref=comm_l.at[A_SLOT[s]],
                dst_ref=comm_l.at[B_SLOT[s]],
                send_sem=send_l.at[s],
                recv_sem=recv_l.at[s],
                device_id=(left,),
                device_id_type=pl.DeviceIdType.MESH,
            )
            return r, l

        comm_r[A_SLOT[0]] = partial_r(my + 3).astype(jnp.bfloat16)
        comm_l[A_SLOT[0]] = partial_l(my + 1).astype(jnp.bfloat16)
        rdma_r, rdma_l = hop_rdmas(0)
        rdma_r.start()
        rdma_l.start()

        for s in range(N_DEV - 1):
            comm_r[C_SLOT[s]] = partial_r(my + 2 - s).astype(jnp.bfloat16)
            comm_l[C_SLOT[s]] = partial_l(my + 2 + s).astype(jnp.bfloat16)

            rdma_r.wait_recv()
            comm_r[C_SLOT[s]] = comm_r[C_SLOT[s]] + comm_r[B_SLOT[s]]
            rdma_l.wait_recv()
            comm_l[C_SLOT[s]] = comm_l[C_SLOT[s]] + comm_l[B_SLOT[s]]
            rdma_r.wait_send()
            rdma_l.wait_send()

            if s < N_DEV - 2:
                rdma_r, rdma_l = hop_rdmas(s + 1)
                rdma_r.start()
                rdma_l.start()

        cp_r = pltpu.make_async_copy(
            comm_r.at[C_SLOT[2]], out_ref.at[:, pl.ds(0, h)], out_sems.at[0]
        )
        cp_l = pltpu.make_async_copy(
            comm_l.at[C_SLOT[2]], out_ref.at[:, pl.ds(h, h)], out_sems.at[1]
        )
        cp_r.start()
        cp_l.start()
        cp_r.wait()
        cp_l.wait()

    return pl.pallas_call(
        body,
        out_shape=jax.ShapeDtypeStruct((m_blk, n), jnp.bfloat16),
        in_specs=[
            pl.BlockSpec(memory_space=pltpu.VMEM),
            pl.BlockSpec(memory_space=pltpu.VMEM),
        ],
        out_specs=pl.BlockSpec(memory_space=pl.ANY),
        scratch_shapes=[
            pltpu.VMEM((3, m_blk, h), jnp.bfloat16),
            pltpu.VMEM((3, m_blk, h), jnp.bfloat16),
            pltpu.SemaphoreType.DMA((3,)),
            pltpu.SemaphoreType.DMA((3,)),
            pltpu.SemaphoreType.DMA((3,)),
            pltpu.SemaphoreType.DMA((3,)),
            pltpu.SemaphoreType.DMA((2,)),
        ],
        compiler_params=pltpu.CompilerParams(
            collective_id=0,
            vmem_limit_bytes=100 * 1024 * 1024,
        ),
    )(A, B)


# device time: 207017 ns/iter; 1.9678x vs baseline; 1.0550x over previous
import jax
import jax.numpy as jnp
from jax import lax
from jax.experimental import pallas as pl
from jax.experimental.pallas import tpu as pltpu

N_DEV = 4
N_LANES = 4

A_SLOT = (0, 2, 1)
B_SLOT = (1, 0, 2)
C_SLOT = (2, 1, 0)

LANES = ((0, +1), (2, -1), (1, +1), (3, -1))


def kernel(A, B):
    A = A.astype(jnp.bfloat16)
    B = B.astype(jnp.bfloat16)
    m, _ = A.shape
    _, n = B.shape
    m_blk = m // N_DEV
    w = n // N_LANES

    def body(a_ref, b_ref, out_ref, comm, send_sems, recv_sems, out_sems):
        my = lax.axis_index("i")
        left = lax.rem(my + N_DEV - 1, N_DEV)
        right = lax.rem(my + 1, N_DEV)

        barrier_sem = pltpu.get_barrier_semaphore()
        for nbr in (left, right):
            pl.semaphore_signal(
                barrier_sem, inc=1,
                device_id=(nbr,), device_id_type=pl.DeviceIdType.MESH,
            )
        pl.semaphore_wait(barrier_sem, 2)

        def a_blk(c):
            return a_ref[pl.ds(lax.rem(c, N_DEV) * m_blk, m_blk), :]

        def lane_dot(c, col):
            return jnp.dot(a_blk(c), b_ref[:, col * w:(col + 1) * w],
                           preferred_element_type=jnp.float32)

        def hop_rdma(li, s):
            col, d = LANES[li]
            return pltpu.make_async_remote_copy(
                src_ref=comm.at[li, A_SLOT[s]],
                dst_ref=comm.at[li, B_SLOT[s]],
                send_sem=send_sems.at[li, s],
                recv_sem=recv_sems.at[li, s],
                device_id=(right,) if d > 0 else (left,),
                device_id_type=pl.DeviceIdType.MESH,
            )

        def recv_chunk(li, s):
            col, d = LANES[li]
            return my + 2 - s if d > 0 else my + 2 + s

        rdmas = [None] * N_LANES

        for li in range(N_LANES):
            col, d = LANES[li]
            c0 = my + 3 if d > 0 else my + 1
            comm[li, A_SLOT[0]] = lane_dot(c0, col).astype(jnp.bfloat16)
            rdmas[li] = hop_rdma(li, 0)
            rdmas[li].start()

        for s in range(N_DEV - 1):
            for li in range(N_LANES):
                col, _ = LANES[li]
                rdmas[li].wait_recv()
                comm[li, C_SLOT[s]] = (
                    lane_dot(recv_chunk(li, s), col)
                    + comm[li, B_SLOT[s]].astype(jnp.float32)
                ).astype(jnp.bfloat16)
                rdmas[li].wait_send()
                if s < N_DEV - 2:
                    rdmas[li] = hop_rdma(li, s + 1)
                    rdmas[li].start()
                else:
                    pltpu.make_async_copy(
                        comm.at[li, C_SLOT[s]],
                        out_ref.at[:, pl.ds(col * w, w)],
                        out_sems.at[li],
                    ).start()

        for li in range(N_LANES):
            pltpu.make_async_copy(
                comm.at[li, C_SLOT[2]],
                out_ref.at[:, pl.ds(LANES[li][0] * w, w)],
                out_sems.at[li],
            ).wait()

    return pl.pallas_call(
        body,
        out_shape=jax.ShapeDtypeStruct((m_blk, n), jnp.bfloat16),
        in_specs=[
            pl.BlockSpec(memory_space=pltpu.VMEM),
            pl.BlockSpec(memory_space=pltpu.VMEM),
        ],
        out_specs=pl.BlockSpec(memory_space=pl.ANY),
        scratch_shapes=[
            pltpu.VMEM((N_LANES, 3, m_blk, w), jnp.bfloat16),
            pltpu.SemaphoreType.DMA((N_LANES, 3)),
            pltpu.SemaphoreType.DMA((N_LANES, 3)),
            pltpu.SemaphoreType.DMA((N_LANES,)),
        ],
        compiler_params=pltpu.CompilerParams(
            collective_id=0,
            vmem_limit_bytes=100 * 1024 * 1024,
        ),
    )(A, B)


# device time: 200324 ns/iter; 2.0335x vs baseline; 1.0334x over previous
import jax
import jax.numpy as jnp
from jax import lax
from jax.experimental import pallas as pl
from jax.experimental.pallas import tpu as pltpu

N_DEV = 4
N_LANES = 8

A_SLOT = (0, 2, 1)
B_SLOT = (1, 0, 2)
C_SLOT = (2, 1, 0)

LANES = ((0, +1), (4, -1), (1, +1), (5, -1),
         (2, +1), (6, -1), (3, +1), (7, -1))


def kernel(A, B):
    A = A.astype(jnp.bfloat16)
    B = B.astype(jnp.bfloat16)
    m, _ = A.shape
    _, n = B.shape
    m_blk = m // N_DEV
    w = n // N_LANES

    def body(a_ref, b_ref, out_ref, comm, send_sems, recv_sems, out_sems):
        my = lax.axis_index("i")
        left = lax.rem(my + N_DEV - 1, N_DEV)
        right = lax.rem(my + 1, N_DEV)

        barrier_sem = pltpu.get_barrier_semaphore()
        for nbr in (left, right):
            pl.semaphore_signal(
                barrier_sem, inc=1,
                device_id=(nbr,), device_id_type=pl.DeviceIdType.MESH,
            )
        pl.semaphore_wait(barrier_sem, 2)

        def a_blk(c):
            return a_ref[pl.ds(lax.rem(c, N_DEV) * m_blk, m_blk), :]

        def lane_dot(c, col):
            return jnp.dot(a_blk(c), b_ref[:, col * w:(col + 1) * w],
                           preferred_element_type=jnp.float32)

        def hop_rdma(li, s):
            col, d = LANES[li]
            return pltpu.make_async_remote_copy(
                src_ref=comm.at[li, A_SLOT[s]],
                dst_ref=comm.at[li, B_SLOT[s]],
                send_sem=send_sems.at[li, s],
                recv_sem=recv_sems.at[li, s],
                device_id=(right,) if d > 0 else (left,),
                device_id_type=pl.DeviceIdType.MESH,
            )

        def recv_chunk(li, s):
            col, d = LANES[li]
            return my + 2 - s if d > 0 else my + 2 + s

        rdmas = [None] * N_LANES

        for li in range(N_LANES):
            col, d = LANES[li]
            c0 = my + 3 if d > 0 else my + 1
            comm[li, A_SLOT[0]] = lane_dot(c0, col).astype(jnp.bfloat16)
            rdmas[li] = hop_rdma(li, 0)
            rdmas[li].start()

        for s in range(N_DEV - 1):
            for li in range(N_LANES):
                col, _ = LANES[li]
                rdmas[li].wait_recv()
                comm[li, C_SLOT[s]] = (
                    lane_dot(recv_chunk(li, s), col)
                    + comm[li, B_SLOT[s]].astype(jnp.float32)
                ).astype(jnp.bfloat16)
                rdmas[li].wait_send()
                if s < N_DEV - 2:
                    rdmas[li] = hop_rdma(li, s + 1)
                    rdmas[li].start()
                else:
                    pltpu.make_async_copy(
                        comm.at[li, C_SLOT[s]],
                        out_ref.at[:, pl.ds(col * w, w)],
                        out_sems.at[li],
                    ).start()

        for li in range(N_LANES):
            pltpu.make_async_copy(
                comm.at[li, C_SLOT[2]],
                out_ref.at[:, pl.ds(LANES[li][0] * w, w)],
                out_sems.at[li],
            ).wait()

    return pl.pallas_call(
        body,
        out_shape=jax.ShapeDtypeStruct((m_blk, n), jnp.bfloat16),
        in_specs=[
            pl.BlockSpec(memory_space=pltpu.VMEM),
            pl.BlockSpec(memory_space=pltpu.VMEM),
        ],
        out_specs=pl.BlockSpec(memory_space=pl.ANY),
        scratch_shapes=[
            pltpu.VMEM((N_LANES, 3, m_blk, w), jnp.bfloat16),
            pltpu.SemaphoreType.DMA((N_LANES, 3)),
            pltpu.SemaphoreType.DMA((N_LANES, 3)),
            pltpu.SemaphoreType.DMA((N_LANES,)),
        ],
        compiler_params=pltpu.CompilerParams(
            collective_id=0,
            vmem_limit_bytes=100 * 1024 * 1024,
        ),
    )(A, B)


# device time: 171994 ns/iter; 2.3684x vs baseline; 1.1647x over previous
import jax
import jax.numpy as jnp
from jax import lax
from jax.experimental import pallas as pl
from jax.experimental.pallas import tpu as pltpu

N_DEV = 4
N_LANES = 8

A_SLOT = (0, 2, 1)
B_SLOT = (1, 0, 2)
C_SLOT = (2, 1, 0)

LANES = ((0, +1), (4, -1), (1, +1), (5, -1),
         (2, +1), (6, -1), (3, +1), (7, -1))

ROLE_OF_RECV = {
    (+1, 0): 2, (-1, 0): 2,
    (+1, 1): 1, (-1, 1): 0,
    (+1, 2): 2, (-1, 2): 2,
}


def kernel(A, B):
    m, k = A.shape
    _, n = B.shape
    m_blk = m // N_DEV
    w = n // N_LANES
    a_half = m_blk // 4

    def body(a_any, b_any, out_ref,
             a16, b16, stg_a, stg_b, comm,
             stg_a_sem, stg_b_sem, send_sems, recv_sems, out_sems):
        my = lax.axis_index("i")
        left = lax.rem(my + N_DEV - 1, N_DEV)
        right = lax.rem(my + 1, N_DEV)

        barrier_sem = pltpu.get_barrier_semaphore()
        for nbr in (left, right):
            pl.semaphore_signal(
                barrier_sem, inc=1,
                device_id=(nbr,), device_id_type=pl.DeviceIdType.MESH,
            )

        def stage_b_start(col):
            pltpu.make_async_copy(
                b_any.at[:, pl.ds(col * w, w)], stg_b, stg_b_sem
            ).start()

        def stage_b_finish(col):
            pltpu.make_async_copy(
                b_any.at[:, pl.ds(col * w, w)], stg_b, stg_b_sem
            ).wait()
            b16[:, pl.ds(col * w, w)] = stg_b[:, :].astype(jnp.bfloat16)

        def stage_a_block(role, blk):
            for half in range(4):
                cp = pltpu.make_async_copy(
                    a_any.at[pl.ds(lax.rem(blk, N_DEV) * m_blk
                                   + half * a_half, a_half), :],
                    stg_a, stg_a_sem,
                )
                cp.start()
                cp.wait()
                a16[role, pl.ds(half * a_half, a_half), :] = (
                    stg_a[:, :].astype(jnp.bfloat16))

        def lane_dot(role, col):
            return jnp.dot(a16[role], b16[:, col * w:(col + 1) * w],
                           preferred_element_type=jnp.float32)

        def hop_rdma(li, s):
            _, d = LANES[li]
            return pltpu.make_async_remote_copy(
                src_ref=comm.at[li, A_SLOT[s]],
                dst_ref=comm.at[li, B_SLOT[s]],
                send_sem=send_sems.at[li, s],
                recv_sem=recv_sems.at[li, s],
                device_id=(right,) if d > 0 else (left,),
                device_id_type=pl.DeviceIdType.MESH,
            )

        stage_b_start(LANES[0][0])
        stage_a_block(0, my + 3)
        stage_a_block(1, my + 1)
        pl.semaphore_wait(barrier_sem, 2)

        rdmas = [None] * N_LANES
        for li in range(N_LANES):
            col, d = LANES[li]
            stage_b_finish(col)
            if li + 1 < N_LANES:
                stage_b_start(LANES[li + 1][0])
            comm[li, A_SLOT[0]] = (
                lane_dot(0 if d > 0 else 1, col).astype(jnp.bfloat16))
            rdmas[li] = hop_rdma(li, 0)
            rdmas[li].start()

        stage_a_block(2, my + 2)

        for s in range(N_DEV - 1):
            for li in range(N_LANES):
                col, d = LANES[li]
                rdmas[li].wait_recv()
                comm[li, C_SLOT[s]] = (
                    lane_dot(ROLE_OF_RECV[(d, s)], col)
                    + comm[li, B_SLOT[s]].astype(jnp.float32)
                ).astype(jnp.bfloat16)
                rdmas[li].wait_send()
                if s < N_DEV - 2:
                    rdmas[li] = hop_rdma(li, s + 1)
                    rdmas[li].start()
                else:
                    pltpu.make_async_copy(
                        comm.at[li, C_SLOT[s]],
                        out_ref.at[:, pl.ds(col * w, w)],
                        out_sems.at[li],
                    ).start()
            if s == 0:
                stage_a_block(2, my)

        for li in range(N_LANES):
            pltpu.make_async_copy(
                comm.at[li, C_SLOT[2]],
                out_ref.at[:, pl.ds(LANES[li][0] * w, w)],
                out_sems.at[li],
            ).wait()

    return pl.pallas_call(
        body,
        out_shape=jax.ShapeDtypeStruct((m_blk, n), jnp.bfloat16),
        in_specs=[
            pl.BlockSpec(memory_space=pl.ANY),
            pl.BlockSpec(memory_space=pl.ANY),
        ],
        out_specs=pl.BlockSpec(memory_space=pl.ANY),
        scratch_shapes=[
            pltpu.VMEM((3, m_blk, k), jnp.bfloat16),
            pltpu.VMEM((k, n), jnp.bfloat16),
            pltpu.VMEM((m_blk // 4, k), jnp.float32),
            pltpu.VMEM((k, n // N_LANES), jnp.float32),
            pltpu.VMEM((N_LANES, 3, m_blk, n // N_LANES), jnp.bfloat16),
            pltpu.SemaphoreType.DMA,
            pltpu.SemaphoreType.DMA,
            pltpu.SemaphoreType.DMA((N_LANES, 3)),
            pltpu.SemaphoreType.DMA((N_LANES, 3)),
            pltpu.SemaphoreType.DMA((N_LANES,)),
        ],
        compiler_params=pltpu.CompilerParams(
            collective_id=0,
            vmem_limit_bytes=100 * 1024 * 1024,
        ),
    )(A, B)
